# baseline (speedup 1.0000x reference)
import functools

import jax
import jax.numpy as jnp
from jax import lax
from jax.experimental import pallas as pl
from jax.experimental.pallas import tpu as pltpu
from jax.experimental.pallas import tpu_sc as plsc

VOCAB = 1000000
ENC = 64
BATCH = 16384

NUM_CORES = 2
NUM_SUBCORES = 16
LANES = 16
NW = NUM_CORES * NUM_SUBCORES
BPW = BATCH // NW
CHUNK = 128
NCHUNK = BPW // CHUNK
ROW = 2 * ENC

_mesh = plsc.VectorSubcoreMesh(core_axis_name="c", subcore_axis_name="s")


@functools.partial(
    pl.kernel,
    mesh=_mesh,
    compiler_params=pltpu.CompilerParams(needs_layout_passes=False),
    out_type=jax.ShapeDtypeStruct((BATCH,), jnp.float32),
    scratch_types=[
        pltpu.VMEM((NCHUNK, CHUNK), jnp.int32),
        pltpu.VMEM((NCHUNK, CHUNK), jnp.int32),
        pltpu.VMEM((BPW,), jnp.int32),
        pltpu.VMEM((BPW,), jnp.int32),
        pltpu.VMEM((BPW // 2, ROW), jnp.float32),
        pltpu.VMEM((BPW // 2, ROW), jnp.float32),
        pltpu.VMEM((BPW,), jnp.float32),
        pltpu.VMEM((4 * LANES * (LANES + 1),), jnp.float32),
        pltpu.SemaphoreType.DMA,
        pltpu.SemaphoreType.DMA,
    ],
)
def _genre2vec_sc(rows_a_hbm, rows_b_hbm, par_a_hbm, par_b_hbm,
                  emb_hbm, ctx_hbm, out_hbm,
                  ia_v, ib_v, pa_v, pb_v, ra_v, rb_v, o_v, ps_v,
                  sem_a, sem_b):
    wid = lax.axis_index("s") * NUM_CORES + lax.axis_index("c")
    base = wid * BPW

    pltpu.sync_copy(rows_a_hbm.at[wid], ia_v)
    pltpu.sync_copy(rows_b_hbm.at[wid], ib_v)
    pltpu.sync_copy(par_a_hbm.at[wid], pa_v)
    pltpu.sync_copy(par_b_hbm.at[wid], pb_v)

    lane_iota = lax.iota(jnp.int32, LANES)
    pitch_iota = lane_iota * (LANES + 1)

    for h in range(2):
        copies = []
        for j in range(NCHUNK // 2):
            jj = h * (NCHUNK // 2) + j
            copies.append(pltpu.async_copy(
                emb_hbm.at[ia_v.at[jj]], ra_v.at[pl.ds(j * CHUNK, CHUNK)],
                sem_a))
            copies.append(pltpu.async_copy(
                ctx_hbm.at[ib_v.at[jj]], rb_v.at[pl.ds(j * CHUNK, CHUNK)],
                sem_b))
        for cp in copies:
            cp.wait()

        def group_body(g, _):
            loc0 = g * LANES
            row0 = h * (BPW // 2) + loc0
            for rl in range(LANES):
                r = loc0 + rl
                a0 = [ra_v[r, pl.ds(k * LANES, LANES)] for k in range(4)]
                a1 = [ra_v[r, pl.ds(ENC + k * LANES, LANES)] for k in range(4)]
                b0 = [rb_v[r, pl.ds(k * LANES, LANES)] for k in range(4)]
                b1 = [rb_v[r, pl.ds(ENC + k * LANES, LANES)] for k in range(4)]
                for t, (av, bv) in enumerate(
                        ((a0, b0), (a0, b1), (a1, b0), (a1, b1))):
                    p = (av[0] * bv[0] + av[1] * bv[1]
                         + av[2] * bv[2] + av[3] * bv[3])
                    ps_v[pl.ds((t * LANES + rl) * (LANES + 1), LANES)] = p
            dots = []
            for t in range(4):
                tbase = t * LANES * (LANES + 1)
                acc = plsc.load_gather(ps_v, [pitch_iota + tbase])
                for c in range(1, LANES):
                    acc = acc + plsc.load_gather(
                        ps_v, [pitch_iota + (tbase + c)])
                dots.append(acc)
            pa = pa_v[pl.ds(row0, LANES)]
            pb = pb_v[pl.ds(row0, LANES)]
            a_even = jnp.where(pb == 0, dots[0], dots[1])
            a_odd = jnp.where(pb == 0, dots[2], dots[3])
            d = jnp.where(pa == 0, a_even, a_odd)
            o_v[pl.ds(row0, LANES)] = 1.0 / (1.0 + jnp.exp(-d))
            return 0

        lax.fori_loop(0, BPW // 2 // LANES, group_body, 0)

    pltpu.sync_copy(o_v, out_hbm.at[pl.ds(base, BPW)])


def kernel(input_genres, context_genres, embedding_table, context_table):
    ia = input_genres.astype(jnp.int32)
    ib = context_genres.astype(jnp.int32)
    rows_a = (ia >> 1).reshape(NW, NCHUNK, CHUNK)
    rows_b = (ib >> 1).reshape(NW, NCHUNK, CHUNK)
    par_a = (ia & 1).reshape(NW, BPW)
    par_b = (ib & 1).reshape(NW, BPW)
    emb2 = embedding_table.reshape(VOCAB // 2, ROW)
    ctx2 = context_table.reshape(VOCAB // 2, ROW)
    return _genre2vec_sc(rows_a, rows_b, par_a, par_b, emb2, ctx2)


# speedup vs baseline: 1.0706x; 1.0706x over previous
import functools

import jax
import jax.numpy as jnp
from jax import lax
from jax.experimental import pallas as pl
from jax.experimental.pallas import tpu as pltpu
from jax.experimental.pallas import tpu_sc as plsc

VOCAB = 1000000
ENC = 64
BATCH = 16384

NUM_CORES = 2
NUM_SUBCORES = 16
LANES = 16
NW = NUM_CORES * NUM_SUBCORES
BPW = BATCH // NW
CHUNK = 128
NCHUNK = BPW // CHUNK
PITCH = LANES + 1

_mesh = plsc.VectorSubcoreMesh(core_axis_name="c", subcore_axis_name="s")


@functools.partial(
    pl.kernel,
    mesh=_mesh,
    compiler_params=pltpu.CompilerParams(needs_layout_passes=False),
    out_type=jax.ShapeDtypeStruct((BATCH,), jnp.float32),
    scratch_types=[
        pltpu.VMEM((NCHUNK, CHUNK), jnp.int32),
        pltpu.VMEM((NCHUNK, CHUNK), jnp.int32),
        pltpu.VMEM((BPW // 2, 128), jnp.float32),
        pltpu.VMEM((BPW // 2, 128), jnp.float32),
        pltpu.VMEM((BPW,), jnp.float32),
        pltpu.VMEM((LANES * PITCH,), jnp.float32),
        pltpu.SemaphoreType.DMA,
        pltpu.SemaphoreType.DMA,
    ],
)
def _genre2vec_sc(rows_a_hbm, rows_b_hbm, emb_hbm, ctx_hbm, out_hbm,
                  ia_v, ib_v, ra_v, rb_v, o_v, ps_v, sem_a, sem_b):
    wid = lax.axis_index("s") * NUM_CORES + lax.axis_index("c")
    base = wid * BPW

    pltpu.sync_copy(rows_a_hbm.at[wid], ia_v)
    pltpu.sync_copy(rows_b_hbm.at[wid], ib_v)

    lane_iota = lax.iota(jnp.int32, LANES)
    pitch_iota = lane_iota * PITCH

    for h in range(2):
        copies = []
        for j in range(NCHUNK // 2):
            jj = h * (NCHUNK // 2) + j
            copies.append(pltpu.async_copy(
                emb_hbm.at[ia_v.at[jj]], ra_v.at[pl.ds(j * CHUNK, CHUNK)],
                sem_a))
            copies.append(pltpu.async_copy(
                ctx_hbm.at[ib_v.at[jj]], rb_v.at[pl.ds(j * CHUNK, CHUNK)],
                sem_b))
        for cp in copies:
            cp.wait()

        def group_body(g, _):
            loc0 = g * LANES
            row0 = h * (BPW // 2) + loc0
            for rl in range(LANES):
                r = loc0 + rl
                pr = (ra_v[r, pl.ds(0, 16)] * rb_v[r, pl.ds(0, 16)]
                      + ra_v[r, pl.ds(16, 16)] * rb_v[r, pl.ds(16, 16)]
                      + ra_v[r, pl.ds(32, 16)] * rb_v[r, pl.ds(32, 16)]
                      + ra_v[r, pl.ds(48, 16)] * rb_v[r, pl.ds(48, 16)])
                ps_v[pl.ds(rl * PITCH, LANES)] = pr
            acc = plsc.load_gather(ps_v, [pitch_iota])
            for c in range(1, LANES):
                acc = acc + plsc.load_gather(ps_v, [pitch_iota + c])
            o_v[pl.ds(row0, LANES)] = 1.0 / (1.0 + jnp.exp(-acc))
            return 0

        lax.fori_loop(0, BPW // 2 // LANES, group_body, 0)

    pltpu.sync_copy(o_v, out_hbm.at[pl.ds(base, BPW)])


def kernel(input_genres, context_genres, embedding_table, context_table):
    ia = input_genres.astype(jnp.int32)
    ib = context_genres.astype(jnp.int32)
    rows_a = ia.reshape(NW, NCHUNK, CHUNK)
    rows_b = ib.reshape(NW, NCHUNK, CHUNK)
    emb_p = jnp.pad(embedding_table, ((0, 0), (0, 128 - ENC)))
    ctx_p = jnp.pad(context_table, ((0, 0), (0, 128 - ENC)))
    return _genre2vec_sc(rows_a, rows_b, emb_p, ctx_p)
